# Initial kernel scaffold; baseline (speedup 1.0000x reference)
import jax
import jax.numpy as jnp
from jax.experimental import pallas as pl


def kernel(text, token_embedding, positional_embedding):
    raise NotImplementedError("write your pallas kernel here")


# speedup vs baseline: 5.0869x; 5.0869x over previous
import functools

import jax
import jax.numpy as jnp
from jax import lax
from jax.experimental import pallas as pl
from jax.experimental.pallas import tpu as pltpu
from jax.experimental.pallas import tpu_sc as plsc

VOCAB = 100000
WIDTH = 128
CONTEXT = 200
BATCH = 4096

ROWS = CONTEXT * BATCH
NUM_WORKERS = 32
ROWS_PER_W = ROWS // NUM_WORKERS
CHUNK = 128
NCHUNK = ROWS_PER_W // CHUNK
IDX_ROWS = NCHUNK
VREGS_PER_ROW = WIDTH // 16


def _build_kernel():
    mesh = plsc.VectorSubcoreMesh(core_axis_name="c", subcore_axis_name="s")

    @functools.partial(
        pl.kernel,
        mesh=mesh,
        out_type=jax.ShapeDtypeStruct((ROWS, WIDTH), jnp.float32),
        scratch_types=[
            pltpu.VMEM((IDX_ROWS, CHUNK), jnp.int32),
            pltpu.VMEM((CONTEXT, WIDTH), jnp.float32),
            pltpu.VMEM((CHUNK, WIDTH), jnp.float32),
            pltpu.SemaphoreType.DMA,
        ],
    )
    def body(idx_hbm, table_hbm, pos_hbm, out_hbm, idx_v, pos_v, rows_v, sem):
        wid = lax.axis_index("s") * 2 + lax.axis_index("c")
        base_row = wid * ROWS_PER_W
        pltpu.sync_copy(idx_hbm.at[pl.ds(wid * IDX_ROWS, IDX_ROWS)], idx_v)
        pltpu.sync_copy(pos_hbm, pos_v)

        def chunk_body(g, carry):
            row0 = base_row + g * CHUNK
            pltpu.async_copy(table_hbm.at[idx_v.at[g]], rows_v, sem).wait()
            l = row0 // BATCH
            pks = [pos_v[l, pl.ds(16 * k, 16)] for k in range(VREGS_PER_ROW)]

            def add_row(j, c):
                for k in range(VREGS_PER_ROW):
                    sl = pl.ds(16 * k, 16)
                    rows_v[j, sl] = rows_v[j, sl] + pks[k]
                return c

            lax.fori_loop(0, CHUNK, add_row, 0, unroll=2)
            pltpu.sync_copy(rows_v, out_hbm.at[pl.ds(row0, CHUNK)])
            return carry

        lax.fori_loop(0, NCHUNK, chunk_body, 0)

    return body


_sc_kernel = _build_kernel()


def kernel(text, token_embedding, positional_embedding):
    idx = jnp.transpose(text).astype(jnp.int32).reshape(ROWS // CHUNK, CHUNK)
    out = _sc_kernel(idx, token_embedding, positional_embedding)
    return out.reshape(CONTEXT, BATCH, WIDTH)
